# baseline (speedup 1.0000x reference)
import jax
import jax.numpy as jnp
from jax import lax
from jax.experimental import pallas as pl
from jax.experimental.pallas import tpu as pltpu

_E = 6400000
_C = 128
_R = _E // _C
_GRID = 25
_BR = _R // _GRID
_CH = 80
_IR = 2 * _R
_IB = _IR // _GRID

_R0 = (13, 15, 26, 6)
_R1 = (17, 29, 16, 24)
_KS1 = 42
_KS2 = 0x1BD11BDA ^ 42


def _rotl(x, r):
    return (x << jnp.uint32(r)) | (x >> jnp.uint32(32 - r))


def _round(x0, x1, r):
    x0 = x0 + x1
    x1 = x0 ^ _rotl(x1, r)
    return x0, x1


def _threefry_bits(x):
    ks1 = jnp.uint32(_KS1)
    ks2 = jnp.uint32(_KS2)
    x1 = x + ks1
    x0 = x1
    x1 = x0 ^ _rotl(x1, _R0[0])
    for r in _R0[1:]:
        x0, x1 = _round(x0, x1, r)
    x0 = x0 + ks1
    x1 = x1 + jnp.uint32((_KS2 + 1) & 0xFFFFFFFF)
    for r in _R1:
        x0, x1 = _round(x0, x1, r)
    x0 = x0 + ks2
    x1 = x1 + jnp.uint32(2)
    for r in _R0:
        x0, x1 = _round(x0, x1, r)
    x1 = x1 + jnp.uint32(_KS1 + 3)
    for r in _R1:
        x0, x1 = _round(x0, x1, r)
    x0 = x0 + ks1
    x1 = x1 + jnp.uint32((_KS2 + 4) & 0xFFFFFFFF)
    for r in _R0:
        x0, x1 = _round(x0, x1, r)
    x0 = x0 + ks2
    x1 = x1 + jnp.uint32(5)
    return x0 ^ x1


def _body(m_ref, inv_ref, vals_ref, idx_hbm, ovals_ref, oidx_hbm, sem):
    g = pl.program_id(0)
    cp = pltpu.make_async_copy(
        idx_hbm.at[pl.ds(g * _IB, _IB)],
        oidx_hbm.at[pl.ds(g * _IB, _IB)],
        sem,
    )
    cp.start()

    mstar = m_ref[0]
    inv = inv_ref[0]
    row = lax.broadcasted_iota(jnp.uint32, (_CH, _C), 0)
    col = lax.broadcasted_iota(jnp.uint32, (_CH, _C), 1)
    lin = (row << jnp.uint32(7)) + col

    def chunk(k, carry):
        r0 = pl.multiple_of(k * _CH, _CH)
        base = ((g * _BR + k * _CH) * _C).astype(jnp.uint32)
        bits = _threefry_bits(lin + base)
        keep = (bits >> jnp.uint32(9)).astype(jnp.int32) >= mstar
        v = vals_ref[pl.ds(r0, _CH), :]
        ovals_ref[pl.ds(r0, _CH), :] = jnp.where(keep, v * inv, 0.0)
        return carry

    lax.fori_loop(0, _BR // _CH, chunk, 0)
    cp.wait()


def kernel(adj_indices, adj_values, keepRate):
    assert adj_values.shape == (_E,) and adj_indices.shape == (2, _E)
    kr = jnp.asarray(keepRate, jnp.float32)
    inv = (1.0 / kr).reshape(1)
    m0 = jnp.ceil((1.0 - kr) * jnp.float32(1 << 23)).astype(jnp.int32)
    cands = jnp.clip(m0 + jnp.arange(-2, 3, dtype=jnp.int32), 0, 1 << 23)
    passing = (cands.astype(jnp.float32) * jnp.float32(2.0 ** -23) + kr) >= 1.0
    mstar = jnp.min(jnp.where(passing, cands, 1 << 23)).reshape(1)

    vals2 = adj_values.reshape(_R, _C)
    idx2 = adj_indices.reshape(_IR, _C)
    ovals, oidx = pl.pallas_call(
        _body,
        grid=(_GRID,),
        in_specs=[
            pl.BlockSpec(memory_space=pltpu.SMEM),
            pl.BlockSpec(memory_space=pltpu.SMEM),
            pl.BlockSpec((_BR, _C), lambda g: (g, 0)),
            pl.BlockSpec(memory_space=pl.ANY),
        ],
        out_specs=[
            pl.BlockSpec((_BR, _C), lambda g: (g, 0)),
            pl.BlockSpec(memory_space=pl.ANY),
        ],
        out_shape=[
            jax.ShapeDtypeStruct((_R, _C), jnp.float32),
            jax.ShapeDtypeStruct((_IR, _C), jnp.int32),
        ],
        scratch_shapes=[pltpu.SemaphoreType.DMA],
        compiler_params=pltpu.CompilerParams(
            dimension_semantics=("arbitrary",),
        ),
    )(mstar, inv, vals2, idx2)
    return oidx.reshape(2, _E), ovals.reshape(_E)


# speedup vs baseline: 1.0001x; 1.0001x over previous
import jax
import jax.numpy as jnp
from jax import lax
from jax.experimental import pallas as pl
from jax.experimental.pallas import tpu as pltpu

_E = 6400000
_C = 128
_R = _E // _C
_GRID = 25
_BR = _R // _GRID
_CH = 80
_IR = 2 * _R
_IB = _IR // _GRID

_R0 = (13, 15, 26, 6)
_R1 = (17, 29, 16, 24)
_KS1 = 42
_KS2 = 0x1BD11BDA ^ 42


def _rotl(x, r):
    return (x << jnp.uint32(r)) | (x >> jnp.uint32(32 - r))


def _round(x0, x1, r):
    x0 = x0 + x1
    x1 = x0 ^ _rotl(x1, r)
    return x0, x1


def _threefry_bits(x):
    ks1 = jnp.uint32(_KS1)
    ks2 = jnp.uint32(_KS2)
    x1 = x + ks1
    x0 = x1
    x1 = x0 ^ _rotl(x1, _R0[0])
    for r in _R0[1:]:
        x0, x1 = _round(x0, x1, r)
    x0 = x0 + ks1
    x1 = x1 + jnp.uint32((_KS2 + 1) & 0xFFFFFFFF)
    for r in _R1:
        x0, x1 = _round(x0, x1, r)
    x0 = x0 + ks2
    x1 = x1 + jnp.uint32(2)
    for r in _R0:
        x0, x1 = _round(x0, x1, r)
    x1 = x1 + jnp.uint32(_KS1 + 3)
    for r in _R1:
        x0, x1 = _round(x0, x1, r)
    x0 = x0 + ks1
    x1 = x1 + jnp.uint32((_KS2 + 4) & 0xFFFFFFFF)
    for r in _R0:
        x0, x1 = _round(x0, x1, r)
    x0 = x0 + ks2
    x1 = x1 + jnp.uint32(5)
    return x0 ^ x1


def _body(m_ref, inv_ref, vals_ref, idx_hbm, ovals_ref, oidx_hbm, sem):
    g = pl.program_id(0)
    cp = pltpu.make_async_copy(
        idx_hbm.at[pl.ds(g * _IB, _IB)],
        oidx_hbm.at[pl.ds(g * _IB, _IB)],
        sem,
    )
    cp.start()

    mstar = m_ref[0]
    inv = inv_ref[0]
    row = lax.broadcasted_iota(jnp.uint32, (_CH, _C), 0)
    col = lax.broadcasted_iota(jnp.uint32, (_CH, _C), 1)
    lin = (row << jnp.uint32(7)) + col

    gbase = (g * (_BR * _C)).astype(jnp.uint32)
    for k in range(_BR // _CH):
        bits = _threefry_bits(lin + (gbase + jnp.uint32(k * _CH * _C)))
        keep = (bits >> jnp.uint32(9)).astype(jnp.int32) >= mstar
        v = vals_ref[k * _CH:(k + 1) * _CH, :]
        ovals_ref[k * _CH:(k + 1) * _CH, :] = jnp.where(keep, v * inv, 0.0)
    cp.wait()


def kernel(adj_indices, adj_values, keepRate):
    assert adj_values.shape == (_E,) and adj_indices.shape == (2, _E)
    kr = jnp.asarray(keepRate, jnp.float32)
    inv = (1.0 / kr).reshape(1)
    m0 = jnp.ceil((1.0 - kr) * jnp.float32(1 << 23)).astype(jnp.int32)
    cands = jnp.clip(m0 + jnp.arange(-2, 3, dtype=jnp.int32), 0, 1 << 23)
    passing = (cands.astype(jnp.float32) * jnp.float32(2.0 ** -23) + kr) >= 1.0
    mstar = jnp.min(jnp.where(passing, cands, 1 << 23)).reshape(1)

    vals2 = adj_values.reshape(_R, _C)
    idx2 = adj_indices.reshape(_IR, _C)
    ovals, oidx = pl.pallas_call(
        _body,
        grid=(_GRID,),
        in_specs=[
            pl.BlockSpec(memory_space=pltpu.SMEM),
            pl.BlockSpec(memory_space=pltpu.SMEM),
            pl.BlockSpec((_BR, _C), lambda g: (g, 0)),
            pl.BlockSpec(memory_space=pl.ANY),
        ],
        out_specs=[
            pl.BlockSpec((_BR, _C), lambda g: (g, 0)),
            pl.BlockSpec(memory_space=pl.ANY),
        ],
        out_shape=[
            jax.ShapeDtypeStruct((_R, _C), jnp.float32),
            jax.ShapeDtypeStruct((_IR, _C), jnp.int32),
        ],
        scratch_shapes=[pltpu.SemaphoreType.DMA],
        compiler_params=pltpu.CompilerParams(
            dimension_semantics=("arbitrary",),
        ),
    )(mstar, inv, vals2, idx2)
    return oidx.reshape(2, _E), ovals.reshape(_E)


# speedup vs baseline: 6.1319x; 6.1311x over previous
import jax
import jax.numpy as jnp
from jax import lax
from jax.experimental import pallas as pl
from jax.experimental.pallas import tpu as pltpu

_E = 6400000
_C = 128
_R = _E // _C
_GRID = 25
_BR = _R // _GRID
_CH = 80
_IR = 2 * _R
_IB = _IR // _GRID

_R0 = (13, 15, 26, 6)
_R1 = (17, 29, 16, 24)
_KS1 = 42
_KS2 = 0x1BD11BDA ^ 42


def _rotl(x, r):
    return (x << jnp.uint32(r)) | (x >> jnp.uint32(32 - r))


def _round(x0, x1, r):
    x0 = x0 + x1
    x1 = x0 ^ _rotl(x1, r)
    return x0, x1


def _threefry_bits(x):
    ks1 = jnp.uint32(_KS1)
    ks2 = jnp.uint32(_KS2)
    x1 = x + ks1
    x0 = x1
    x1 = x0 ^ _rotl(x1, _R0[0])
    for r in _R0[1:]:
        x0, x1 = _round(x0, x1, r)
    x0 = x0 + ks1
    x1 = x1 + jnp.uint32((_KS2 + 1) & 0xFFFFFFFF)
    for r in _R1:
        x0, x1 = _round(x0, x1, r)
    x0 = x0 + ks2
    x1 = x1 + jnp.uint32(2)
    for r in _R0:
        x0, x1 = _round(x0, x1, r)
    x1 = x1 + jnp.uint32(_KS1 + 3)
    for r in _R1:
        x0, x1 = _round(x0, x1, r)
    x0 = x0 + ks1
    x1 = x1 + jnp.uint32((_KS2 + 4) & 0xFFFFFFFF)
    for r in _R0:
        x0, x1 = _round(x0, x1, r)
    x0 = x0 + ks2
    x1 = x1 + jnp.uint32(5)
    return x0 ^ x1


def _body(m_ref, inv_ref, vals_ref, idx_ref, ovals_ref, oidx_ref):
    g = pl.program_id(0)
    oidx_ref[...] = idx_ref[...]

    mstar = m_ref[0]
    inv = inv_ref[0]
    row = lax.broadcasted_iota(jnp.uint32, (_CH, _C), 0)
    col = lax.broadcasted_iota(jnp.uint32, (_CH, _C), 1)
    lin = (row << jnp.uint32(7)) + col

    gbase = (g * (_BR * _C)).astype(jnp.uint32)
    for k in range(_BR // _CH):
        bits = _threefry_bits(lin + (gbase + jnp.uint32(k * _CH * _C)))
        keep = (bits >> jnp.uint32(9)).astype(jnp.int32) >= mstar
        v = vals_ref[k * _CH:(k + 1) * _CH, :]
        ovals_ref[k * _CH:(k + 1) * _CH, :] = jnp.where(keep, v * inv, 0.0)


def kernel(adj_indices, adj_values, keepRate):
    assert adj_values.shape == (_E,) and adj_indices.shape == (2, _E)
    kr = jnp.asarray(keepRate, jnp.float32)
    inv = (1.0 / kr).reshape(1)
    m0 = jnp.ceil((1.0 - kr) * jnp.float32(1 << 23)).astype(jnp.int32)
    cands = jnp.clip(m0 + jnp.arange(-2, 3, dtype=jnp.int32), 0, 1 << 23)
    passing = (cands.astype(jnp.float32) * jnp.float32(2.0 ** -23) + kr) >= 1.0
    mstar = jnp.min(jnp.where(passing, cands, 1 << 23)).reshape(1)

    vals2 = adj_values.reshape(_R, _C)
    idx2 = adj_indices.reshape(_IR, _C)
    ovals, oidx = pl.pallas_call(
        _body,
        grid=(_GRID,),
        in_specs=[
            pl.BlockSpec(memory_space=pltpu.SMEM),
            pl.BlockSpec(memory_space=pltpu.SMEM),
            pl.BlockSpec((_BR, _C), lambda g: (g, 0)),
            pl.BlockSpec((_IB, _C), lambda g: (g, 0)),
        ],
        out_specs=[
            pl.BlockSpec((_BR, _C), lambda g: (g, 0)),
            pl.BlockSpec((_IB, _C), lambda g: (g, 0)),
        ],
        out_shape=[
            jax.ShapeDtypeStruct((_R, _C), jnp.float32),
            jax.ShapeDtypeStruct((_IR, _C), jnp.int32),
        ],
        compiler_params=pltpu.CompilerParams(
            dimension_semantics=("arbitrary",),
        ),
    )(mstar, inv, vals2, idx2)
    return oidx.reshape(2, _E), ovals.reshape(_E)


# speedup vs baseline: 13.3132x; 2.1711x over previous
import jax
import jax.numpy as jnp
from jax import lax
from jax.experimental import pallas as pl
from jax.experimental.pallas import tpu as pltpu

_E = 6400000
_C = 128
_R = _E // _C
_GRID = 25
_BR = _R // _GRID
_CH = 80
_IR = 2 * _R
_IB = _IR // _GRID

_R0 = (13, 15, 26, 6)
_R1 = (17, 29, 16, 24)
_KS1 = 42
_KS2 = 0x1BD11BDA ^ 42


def _rotl(x, r):
    return (x << jnp.uint32(r)) | (x >> jnp.uint32(32 - r))


def _round(x0, x1, r):
    x0 = x0 + x1
    x1 = x0 ^ _rotl(x1, r)
    return x0, x1


def _threefry_bits(x):
    ks1 = jnp.uint32(_KS1)
    ks2 = jnp.uint32(_KS2)
    x1 = x + ks1
    x0 = x1
    x1 = x0 ^ _rotl(x1, _R0[0])
    for r in _R0[1:]:
        x0, x1 = _round(x0, x1, r)
    x0 = x0 + ks1
    x1 = x1 + jnp.uint32((_KS2 + 1) & 0xFFFFFFFF)
    for r in _R1:
        x0, x1 = _round(x0, x1, r)
    x0 = x0 + ks2
    x1 = x1 + jnp.uint32(2)
    for r in _R0:
        x0, x1 = _round(x0, x1, r)
    x1 = x1 + jnp.uint32(_KS1 + 3)
    for r in _R1:
        x0, x1 = _round(x0, x1, r)
    x0 = x0 + ks1
    x1 = x1 + jnp.uint32((_KS2 + 4) & 0xFFFFFFFF)
    for r in _R0:
        x0, x1 = _round(x0, x1, r)
    x0 = x0 + ks2
    x1 = x1 + jnp.uint32(5)
    return x0 ^ x1


def _body(m_ref, inv_ref, vals_ref, ovals_ref):
    g = pl.program_id(0)
    mstar = m_ref[0]
    inv = inv_ref[0]
    row = lax.broadcasted_iota(jnp.uint32, (_CH, _C), 0)
    col = lax.broadcasted_iota(jnp.uint32, (_CH, _C), 1)
    lin = (row << jnp.uint32(7)) + col

    gbase = (g * (_BR * _C)).astype(jnp.uint32)
    for k in range(_BR // _CH):
        bits = _threefry_bits(lin + (gbase + jnp.uint32(k * _CH * _C)))
        keep = (bits >> jnp.uint32(9)).astype(jnp.int32) >= mstar
        v = vals_ref[k * _CH:(k + 1) * _CH, :]
        ovals_ref[k * _CH:(k + 1) * _CH, :] = jnp.where(keep, v * inv, 0.0)


def kernel(adj_indices, adj_values, keepRate):
    assert adj_values.shape == (_E,) and adj_indices.shape == (2, _E)
    kr = jnp.asarray(keepRate, jnp.float32)
    inv = (1.0 / kr).reshape(1)
    m0 = jnp.ceil((1.0 - kr) * jnp.float32(1 << 23)).astype(jnp.int32)
    cands = jnp.clip(m0 + jnp.arange(-2, 3, dtype=jnp.int32), 0, 1 << 23)
    passing = (cands.astype(jnp.float32) * jnp.float32(2.0 ** -23) + kr) >= 1.0
    mstar = jnp.min(jnp.where(passing, cands, 1 << 23)).reshape(1)

    vals2 = adj_values.reshape(_R, _C)
    ovals = pl.pallas_call(
        _body,
        grid=(_GRID,),
        in_specs=[
            pl.BlockSpec(memory_space=pltpu.SMEM),
            pl.BlockSpec(memory_space=pltpu.SMEM),
            pl.BlockSpec((_BR, _C), lambda g: (g, 0)),
        ],
        out_specs=pl.BlockSpec((_BR, _C), lambda g: (g, 0)),
        out_shape=jax.ShapeDtypeStruct((_R, _C), jnp.float32),
        compiler_params=pltpu.CompilerParams(
            dimension_semantics=("arbitrary",),
        ),
    )(mstar, inv, vals2)
    return adj_indices, ovals.reshape(_E)
